# Initial kernel scaffold; baseline (speedup 1.0000x reference)
import jax
import jax.numpy as jnp
from jax.experimental import pallas as pl


def kernel(vis_feats, vis_pe, input_ids, word_emb, pos_emb, type_emb, ln_gamma, ln_beta):
    raise NotImplementedError("write your pallas kernel here")


# speedup vs baseline: 2.3571x; 2.3571x over previous
import jax
import jax.numpy as jnp
from jax import lax
from jax.experimental import pallas as pl
from jax.experimental.pallas import tpu as pltpu
import jax.experimental.pallas.tpu_sc as plsc

VOCAB = 30522
HID = 768
B = 32
S = 512
LV = 100
EPS = 1e-5

GCHUNK = 32
NCHUNK = S // GCHUNK
VCHUNK = 32
LANES = 16
HID_V = HID // LANES
SPLICE = 128


def _sc_gather(ids_hbm, word_hbm, visf_hbm, vispe_hbm, out_hbm, vs_hbm,
               idx_v, rows_v, visa_v, visb_v, sem):
  wid = lax.axis_index("s") * 2 + lax.axis_index("c")
  base = wid * S
  for c in range(NCHUNK):
    pltpu.sync_copy(ids_hbm.at[pl.ds(base + c * GCHUNK, GCHUNK)], idx_v)
    pltpu.async_copy(word_hbm.at[idx_v], rows_v, sem).wait()
    pltpu.sync_copy(rows_v, out_hbm.at[pl.ds(base + c * GCHUNK, GCHUNK)])
  pltpu.sync_copy(visf_hbm.at[wid], visa_v)
  for c in range(4):
    n = VCHUNK if c < 3 else LV - 3 * VCHUNK
    pltpu.sync_copy(vispe_hbm.at[wid, pl.ds(c * VCHUNK, n)],
                    visb_v.at[pl.ds(0, n)])

    def add_row(r, carry):
      for k in range(HID_V):
        sl = pl.ds(k * LANES, LANES)
        visa_v[c * VCHUNK + r, sl] = visa_v[c * VCHUNK + r, sl] + visb_v[r, sl]
      return carry

    lax.fori_loop(0, n, add_row, 0)
  pltpu.sync_copy(visa_v, vs_hbm.at[wid])


def _tc_body(a_ref, vs_ref, pos_ref, typ_ref, gam_ref, bet_ref, o_ref):
  a = a_ref[0]
  vs = vs_ref[0]
  r = lax.broadcasted_iota(jnp.int32, (SPLICE, 1), 0)
  c = lax.broadcasted_iota(jnp.int32, (1, LV), 1)
  perm = (r == c + 1).astype(jnp.float32)
  vss = jax.lax.dot(perm, vs,
                    preferred_element_type=jnp.float32)
  in_vis = jnp.logical_and(r >= 1, r <= LV)
  head = jnp.where(in_vis, vss,
                   a[:SPLICE] + pos_ref[:SPLICE])
  tail = a[SPLICE:] + pos_ref[SPLICE:]
  emb = jnp.concatenate([head, tail], axis=0) + typ_ref[...]
  u = jnp.mean(emb, axis=1, keepdims=True)
  d = emb - u
  var = jnp.mean(d * d, axis=1, keepdims=True)
  x = d * lax.rsqrt(var + EPS)
  o_ref[0] = gam_ref[...] * x + bet_ref[...]


@jax.jit
def kernel(vis_feats, vis_pe, input_ids, word_emb, pos_emb, type_emb,
           ln_gamma, ln_beta):
  ids = input_ids.reshape(-1).astype(jnp.int32)

  mesh = plsc.VectorSubcoreMesh(core_axis_name="c", subcore_axis_name="s")
  gathered, vsum = pl.kernel(
      _sc_gather,
      out_type=(
          jax.ShapeDtypeStruct((B * S, HID), jnp.float32),
          jax.ShapeDtypeStruct((B, LV, HID), jnp.float32),
      ),
      mesh=mesh,
      scratch_types=[
          pltpu.VMEM((GCHUNK,), jnp.int32),
          pltpu.VMEM((GCHUNK, HID), jnp.float32),
          pltpu.VMEM((LV, HID), jnp.float32),
          pltpu.VMEM((VCHUNK, HID), jnp.float32),
          pltpu.SemaphoreType.DMA,
      ],
  )(ids, word_emb, vis_feats, vis_pe)

  a3 = gathered.reshape(B, S, HID)
  out = pl.pallas_call(
      _tc_body,
      grid=(B,),
      in_specs=[
          pl.BlockSpec((1, S, HID), lambda b: (b, 0, 0)),
          pl.BlockSpec((1, LV, HID), lambda b: (b, 0, 0)),
          pl.BlockSpec((S, HID), lambda b: (0, 0)),
          pl.BlockSpec((1, HID), lambda b: (0, 0)),
          pl.BlockSpec((1, HID), lambda b: (0, 0)),
          pl.BlockSpec((1, HID), lambda b: (0, 0)),
      ],
      out_specs=pl.BlockSpec((1, S, HID), lambda b: (b, 0, 0)),
      out_shape=jax.ShapeDtypeStruct((B, S, HID), jnp.float32),
  )(a3, vsum, pos_emb, type_emb[0:1], ln_gamma.reshape(1, HID),
    ln_beta.reshape(1, HID))
  return out


# speedup vs baseline: 3.3060x; 1.4026x over previous
import jax
import jax.numpy as jnp
from jax import lax
from jax.experimental import pallas as pl
from jax.experimental.pallas import tpu as pltpu
import jax.experimental.pallas.tpu_sc as plsc

VOCAB = 30522
HID = 768
B = 32
S = 512
LV = 100
EPS = 1e-5

GCHUNK = 32
NCHUNK = S // GCHUNK
SPLICE = 128
_CHUNKS = [c for c in range(NCHUNK)
           if not (c * GCHUNK >= 1 and (c + 1) * GCHUNK - 1 <= LV)]


def _sc_gather(ids_hbm, word_hbm, out_hbm, idx_v, rows_v, sem):
  wid = lax.axis_index("s") * 2 + lax.axis_index("c")
  base = wid * S
  for c in _CHUNKS:
    pltpu.sync_copy(ids_hbm.at[pl.ds(base + c * GCHUNK, GCHUNK)], idx_v)
    pltpu.async_copy(word_hbm.at[idx_v], rows_v, sem).wait()
    pltpu.sync_copy(rows_v, out_hbm.at[pl.ds(base + c * GCHUNK, GCHUNK)])


def _tc_body(a_ref, visf_ref, vispe_ref, pos_ref, typ_ref, gam_ref, bet_ref,
             o_ref):
  a = a_ref[0]
  vs = visf_ref[0] + vispe_ref[0]
  r = lax.broadcasted_iota(jnp.int32, (SPLICE, 1), 0)
  c = lax.broadcasted_iota(jnp.int32, (1, LV), 1)
  perm = (r == c + 1).astype(jnp.float32)
  vss = jax.lax.dot(perm, vs,
                    preferred_element_type=jnp.float32)
  in_vis = jnp.logical_and(r >= 1, r <= LV)
  head = jnp.where(in_vis, vss,
                   a[:SPLICE] + pos_ref[:SPLICE])
  tail = a[SPLICE:] + pos_ref[SPLICE:]
  emb = jnp.concatenate([head, tail], axis=0) + typ_ref[...]
  u = jnp.mean(emb, axis=1, keepdims=True)
  d = emb - u
  var = jnp.mean(d * d, axis=1, keepdims=True)
  x = d * lax.rsqrt(var + EPS)
  o_ref[0] = gam_ref[...] * x + bet_ref[...]


@jax.jit
def kernel(vis_feats, vis_pe, input_ids, word_emb, pos_emb, type_emb,
           ln_gamma, ln_beta):
  ids = input_ids.reshape(-1).astype(jnp.int32)

  mesh = plsc.VectorSubcoreMesh(core_axis_name="c", subcore_axis_name="s")
  gathered = pl.kernel(
      _sc_gather,
      out_type=jax.ShapeDtypeStruct((B * S, HID), jnp.float32),
      mesh=mesh,
      scratch_types=[
          pltpu.VMEM((GCHUNK,), jnp.int32),
          pltpu.VMEM((GCHUNK, HID), jnp.float32),
          pltpu.SemaphoreType.DMA,
      ],
  )(ids, word_emb)

  a3 = gathered.reshape(B, S, HID)
  out = pl.pallas_call(
      _tc_body,
      grid=(B,),
      in_specs=[
          pl.BlockSpec((1, S, HID), lambda b: (b, 0, 0)),
          pl.BlockSpec((1, LV, HID), lambda b: (b, 0, 0)),
          pl.BlockSpec((1, LV, HID), lambda b: (b, 0, 0)),
          pl.BlockSpec((S, HID), lambda b: (0, 0)),
          pl.BlockSpec((1, HID), lambda b: (0, 0)),
          pl.BlockSpec((1, HID), lambda b: (0, 0)),
          pl.BlockSpec((1, HID), lambda b: (0, 0)),
      ],
      out_specs=pl.BlockSpec((1, S, HID), lambda b: (b, 0, 0)),
      out_shape=jax.ShapeDtypeStruct((B, S, HID), jnp.float32),
  )(a3, vis_feats, vis_pe, pos_emb, type_emb[0:1], ln_gamma.reshape(1, HID),
    ln_beta.reshape(1, HID))
  return out


# speedup vs baseline: 3.5466x; 1.0728x over previous
import jax
import jax.numpy as jnp
from jax import lax
from jax.experimental import pallas as pl
from jax.experimental.pallas import tpu as pltpu
import jax.experimental.pallas.tpu_sc as plsc

VOCAB = 30522
HID = 768
B = 32
S = 512
LV = 100
EPS = 1e-5

HB = B // 2
GCHUNK = 32
NCHUNK = S // GCHUNK
SPLICE = 128
_CHUNKS = [c for c in range(NCHUNK)
           if not (c * GCHUNK >= 1 and (c + 1) * GCHUNK - 1 <= LV)]
_NC2 = len(_CHUNKS) // 2


def _sc_gather(ids_hbm, word_hbm, out_hbm, idx_v, rows_v, sem):
  wid = lax.axis_index("s") * 2 + lax.axis_index("c")
  b_loc = wid // 2
  for i in range(_NC2):
    c0 = _CHUNKS[i]
    c1 = _CHUNKS[_NC2 + i]
    c = jnp.where(wid % 2 == 0, c0, c1)
    base = b_loc * S + c * GCHUNK
    pltpu.sync_copy(ids_hbm.at[pl.ds(base, GCHUNK)], idx_v)
    pltpu.async_copy(word_hbm.at[idx_v], rows_v, sem).wait()
    pltpu.sync_copy(rows_v, out_hbm.at[pl.ds(base, GCHUNK)])


def _tc_body(dummy_ref, a_ref, visf_ref, vispe_ref, pos_ref, typ_ref,
             gam_ref, bet_ref, o_ref):
  a = a_ref[0]
  vs = visf_ref[0] + vispe_ref[0]
  r = lax.broadcasted_iota(jnp.int32, (SPLICE, 1), 0)
  c = lax.broadcasted_iota(jnp.int32, (1, LV), 1)
  perm = (r == c + 1).astype(jnp.float32)
  vss = jax.lax.dot(perm, vs,
                    preferred_element_type=jnp.float32)
  in_vis = jnp.logical_and(r >= 1, r <= LV)
  head = jnp.where(in_vis, vss,
                   a[:SPLICE] + pos_ref[:SPLICE])
  tail = a[SPLICE:] + pos_ref[SPLICE:]
  emb = jnp.concatenate([head, tail], axis=0) + typ_ref[...]
  u = jnp.mean(emb, axis=1, keepdims=True)
  d = emb - u
  var = jnp.mean(d * d, axis=1, keepdims=True)
  x = d * lax.rsqrt(var + EPS)
  o_ref[0] = gam_ref[...] * x + bet_ref[...]


def _sc_call(ids_half, word_emb):
  mesh = plsc.VectorSubcoreMesh(core_axis_name="c", subcore_axis_name="s")
  return pl.kernel(
      _sc_gather,
      out_type=jax.ShapeDtypeStruct((HB * S, HID), jnp.float32),
      mesh=mesh,
      scratch_types=[
          pltpu.VMEM((GCHUNK,), jnp.int32),
          pltpu.VMEM((GCHUNK, HID), jnp.float32),
          pltpu.SemaphoreType.DMA,
      ],
  )(ids_half, word_emb)


def _tc_call(dummy, a_half, off, vis_feats, vis_pe, pos_emb, typ, gam, bet,
             alias):
  return pl.pallas_call(
      _tc_body,
      grid=(HB,),
      in_specs=[
          pl.BlockSpec(memory_space=pl.ANY),
          pl.BlockSpec((1, S, HID), lambda b: (b, 0, 0)),
          pl.BlockSpec((1, LV, HID), lambda b, o=off: (b + o, 0, 0)),
          pl.BlockSpec((1, LV, HID), lambda b, o=off: (b + o, 0, 0)),
          pl.BlockSpec((S, HID), lambda b: (0, 0)),
          pl.BlockSpec((1, HID), lambda b: (0, 0)),
          pl.BlockSpec((1, HID), lambda b: (0, 0)),
          pl.BlockSpec((1, HID), lambda b: (0, 0)),
      ],
      out_specs=pl.BlockSpec((1, S, HID), lambda b, o=off: (b + o, 0, 0)),
      out_shape=jax.ShapeDtypeStruct((B, S, HID), jnp.float32),
      input_output_aliases={0: 0} if alias else {},
  )(dummy, a_half, vis_feats, vis_pe, pos_emb, typ, gam, bet)


@jax.jit
def kernel(vis_feats, vis_pe, input_ids, word_emb, pos_emb, type_emb,
           ln_gamma, ln_beta):
  ids = input_ids.reshape(-1).astype(jnp.int32)
  ids1, ids2 = ids[:HB * S], ids[HB * S:]

  g1 = _sc_call(ids1, word_emb)
  g2 = _sc_call(ids2, word_emb)

  a1 = g1.reshape(HB, S, HID)
  a2 = g2.reshape(HB, S, HID)
  typ = type_emb[0:1]
  gam = ln_gamma.reshape(1, HID)
  bet = ln_beta.reshape(1, HID)

  out1 = _tc_call(a1, a1, 0, vis_feats, vis_pe, pos_emb, typ, gam, bet,
                  alias=False)
  out2 = _tc_call(out1, a2, HB, vis_feats, vis_pe, pos_emb, typ, gam, bet,
                  alias=True)
  return out2


# speedup vs baseline: 3.7910x; 1.0689x over previous
import jax
import jax.numpy as jnp
from jax import lax
from jax.experimental import pallas as pl
from jax.experimental.pallas import tpu as pltpu
import jax.experimental.pallas.tpu_sc as plsc

VOCAB = 30522
HID = 768
B = 32
S = 512
LV = 100
EPS = 1e-5

NSPLIT = 4
QB = B // NSPLIT
GCHUNK = 32
SPLICE = 128
NCH = 14
SC_ROWS = NCH * GCHUNK


def _sc_gather(ids_hbm, word_hbm, out_hbm, idx_a, idx_b, rows_a, rows_b,
               gsem_a, gsem_b, ssem_a, ssem_b):
  wid = lax.axis_index("s") * 2 + lax.axis_index("c")
  b_loc = wid // NSPLIT
  q = wid % NSPLIT
  start = q * 4 - jnp.maximum(q - 2, 0)
  last = q * 4 + 3 - jnp.maximum(q - 1, 0)

  idx_v = (idx_a, idx_b)
  rows_v = (rows_a, rows_b)
  gsem = (gsem_a, gsem_b)
  ssem = (ssem_a, ssem_b)
  gd = [None, None]
  st = [None, None]
  for i in range(4):
    p = i % 2
    if st[p] is not None:
      st[p].wait()
    ci = jnp.minimum(start + i, last)
    c = ci + 2 * (ci >= 1).astype(jnp.int32)
    pltpu.sync_copy(ids_hbm.at[pl.ds(b_loc * S + c * GCHUNK, GCHUNK)],
                    idx_v[p])
    gd[p] = pltpu.async_copy(word_hbm.at[idx_v[p]], rows_v[p], gsem[p])
    pp = (i - 1) % 2
    if gd[pp] is not None and i >= 1:
      gd[pp].wait()
      st[pp] = pltpu.async_copy(
          rows_v[pp],
          out_hbm.at[pl.ds(b_loc * SC_ROWS + _ci_prev(start, last, i - 1)
                           * GCHUNK, GCHUNK)],
          ssem[pp])
  gd[3 % 2].wait()
  st[3 % 2] = pltpu.async_copy(
      rows_v[3 % 2],
      out_hbm.at[pl.ds(b_loc * SC_ROWS + _ci_prev(start, last, 3)
                       * GCHUNK, GCHUNK)],
      ssem[3 % 2])
  st[0].wait()
  st[1].wait()


def _ci_prev(start, last, i):
  return jnp.minimum(start + i, last)


def _tc_body(dummy_ref, a_ref, visf_ref, vispe_ref, pos_ref, typ_ref,
             gam_ref, bet_ref, o_ref):
  b = pl.program_id(0)
  a = a_ref[0]
  vs = visf_ref[:, b, :] + vispe_ref[:, b, :]
  r = lax.broadcasted_iota(jnp.int32, (SPLICE, 1), 0)
  c = lax.broadcasted_iota(jnp.int32, (1, LV), 1)
  perm = (r == c + 1).astype(jnp.float32)
  vss = jax.lax.dot(perm, vs,
                    preferred_element_type=jnp.float32)
  in_vis = jnp.logical_and(r >= 1, r <= LV)
  head_src = jnp.concatenate([a[0:32], a[0:32], a[0:32], a[32:64]], axis=0)
  head = jnp.where(in_vis, vss, head_src + pos_ref[:SPLICE])
  tail = a[64:SC_ROWS] + pos_ref[SPLICE:]
  emb = jnp.concatenate([head, tail], axis=0) + typ_ref[...]
  u = jnp.mean(emb, axis=1, keepdims=True)
  d = emb - u
  var = jnp.mean(d * d, axis=1, keepdims=True)
  x = d * lax.rsqrt(var + EPS)
  o_ref[0] = gam_ref[...] * x + bet_ref[...]


def _sc_call(ids_q, word_emb):
  mesh = plsc.VectorSubcoreMesh(core_axis_name="c", subcore_axis_name="s")
  return pl.kernel(
      _sc_gather,
      out_type=jax.ShapeDtypeStruct((QB * SC_ROWS, HID), jnp.float32),
      mesh=mesh,
      scratch_types=[
          pltpu.VMEM((GCHUNK,), jnp.int32),
          pltpu.VMEM((GCHUNK,), jnp.int32),
          pltpu.VMEM((GCHUNK, HID), jnp.float32),
          pltpu.VMEM((GCHUNK, HID), jnp.float32),
          pltpu.SemaphoreType.DMA,
          pltpu.SemaphoreType.DMA,
          pltpu.SemaphoreType.DMA,
          pltpu.SemaphoreType.DMA,
      ],
  )(ids_q, word_emb)


def _tc_call(dummy, a_q, qi, vt_f, vt_p, pos_emb, typ, gam, bet, alias):
  return pl.pallas_call(
      _tc_body,
      grid=(QB,),
      in_specs=[
          pl.BlockSpec(memory_space=pl.ANY),
          pl.BlockSpec((1, SC_ROWS, HID), lambda b: (b, 0, 0)),
          pl.BlockSpec((LV, QB, HID), lambda b, q=qi: (0, q, 0)),
          pl.BlockSpec((LV, QB, HID), lambda b, q=qi: (0, q, 0)),
          pl.BlockSpec((S, HID), lambda b: (0, 0)),
          pl.BlockSpec((1, HID), lambda b: (0, 0)),
          pl.BlockSpec((1, HID), lambda b: (0, 0)),
          pl.BlockSpec((1, HID), lambda b: (0, 0)),
      ],
      out_specs=pl.BlockSpec((1, S, HID), lambda b, q=qi: (b + q * QB, 0, 0)),
      out_shape=jax.ShapeDtypeStruct((B, S, HID), jnp.float32),
      input_output_aliases={0: 0} if alias else {},
  )(dummy, a_q, vt_f, vt_p, pos_emb, typ, gam, bet)


@jax.jit
def kernel(vis_feats, vis_pe, input_ids, word_emb, pos_emb, type_emb,
           ln_gamma, ln_beta):
  ids = input_ids.reshape(-1).astype(jnp.int32)

  gs = [_sc_call(ids[qi * QB * S:(qi + 1) * QB * S], word_emb)
        for qi in range(NSPLIT)]

  typ = type_emb[0:1]
  gam = ln_gamma.reshape(1, HID)
  bet = ln_beta.reshape(1, HID)
  vt_f = vis_feats.transpose(1, 0, 2)
  vt_p = vis_pe.transpose(1, 0, 2)

  out = None
  for qi in range(NSPLIT):
    a_q = gs[qi].reshape(QB, SC_ROWS, HID)
    dummy = a_q if out is None else out
    out = _tc_call(dummy, a_q, qi, vt_f, vt_p, pos_emb, typ, gam, bet,
                   alias=out is not None)
  return out


# speedup vs baseline: 3.7952x; 1.0011x over previous
import jax
import jax.numpy as jnp
from jax import lax
from jax.experimental import pallas as pl
from jax.experimental.pallas import tpu as pltpu
import jax.experimental.pallas.tpu_sc as plsc

VOCAB = 30522
HID = 768
B = 32
S = 512
LV = 100
EPS = 1e-5

NSPLIT = 4
QB = B // NSPLIT
GCHUNK = 32
SPLICE = 128
NCH = 14
SC_ROWS = NCH * GCHUNK


def _sc_gather(ids_hbm, word_hbm, out_hbm, idx_a, idx_b, rows_a, rows_b,
               gsem_a, gsem_b, ssem_a, ssem_b):
  wid = lax.axis_index("s") * 2 + lax.axis_index("c")
  b_loc = wid // NSPLIT
  q = wid % NSPLIT
  start = q * 4 - jnp.maximum(q - 2, 0)
  last = q * 4 + 3 - jnp.maximum(q - 1, 0)

  idx_v = (idx_a, idx_b)
  rows_v = (rows_a, rows_b)
  gsem = (gsem_a, gsem_b)
  ssem = (ssem_a, ssem_b)
  gd = [None, None]
  st = [None, None]
  for i in range(4):
    p = i % 2
    if st[p] is not None:
      st[p].wait()
    ci = jnp.minimum(start + i, last)
    c = ci + 2 * (ci >= 1).astype(jnp.int32)
    pltpu.sync_copy(ids_hbm.at[pl.ds(b_loc * S + c * GCHUNK, GCHUNK)],
                    idx_v[p])
    gd[p] = pltpu.async_copy(word_hbm.at[idx_v[p]], rows_v[p], gsem[p])
    pp = (i - 1) % 2
    if gd[pp] is not None and i >= 1:
      gd[pp].wait()
      st[pp] = pltpu.async_copy(
          rows_v[pp],
          out_hbm.at[pl.ds(b_loc * SC_ROWS + _ci_prev(start, last, i - 1)
                           * GCHUNK, GCHUNK)],
          ssem[pp])
  gd[3 % 2].wait()
  st[3 % 2] = pltpu.async_copy(
      rows_v[3 % 2],
      out_hbm.at[pl.ds(b_loc * SC_ROWS + _ci_prev(start, last, 3)
                       * GCHUNK, GCHUNK)],
      ssem[3 % 2])
  st[0].wait()
  st[1].wait()


def _ci_prev(start, last, i):
  return jnp.minimum(start + i, last)


def _ln(emb, gam, bet):
  u = jnp.mean(emb, axis=1, keepdims=True)
  d = emb - u
  var = jnp.mean(d * d, axis=1, keepdims=True)
  return gam * (d * lax.rsqrt(var + EPS)) + bet


def _tc_body(dummy_ref, a_ref, visf_ref, vispe_ref, pos_ref, typ_ref,
             gam_ref, bet_ref, o_ref):
  b = pl.program_id(0)
  a = a_ref[0]
  typ = typ_ref[...]
  gam = gam_ref[...]
  bet = bet_ref[...]
  vs = visf_ref[:, b, :] + vispe_ref[:, b, :]
  r = lax.broadcasted_iota(jnp.int32, (SPLICE, 1), 0)
  c = lax.broadcasted_iota(jnp.int32, (1, LV), 1)
  perm = (r == c + 1).astype(jnp.float32)
  vss = jax.lax.dot(perm, vs,
                    preferred_element_type=jnp.float32)
  m0 = jnp.logical_and(r[:32] >= 1, r[:32] <= LV)
  seg0 = jnp.where(m0, vss[:32], a[0:32] + pos_ref[0:32]) + typ
  o_ref[0, 0:32, :] = _ln(seg0, gam, bet)
  seg1 = vss[32:96] + typ
  o_ref[0, 32:96, :] = _ln(seg1, gam, bet)
  m2 = r[96:SPLICE] <= LV
  seg2 = jnp.where(m2, vss[96:SPLICE], a[32:64] + pos_ref[96:SPLICE]) + typ
  o_ref[0, 96:SPLICE, :] = _ln(seg2, gam, bet)
  seg3 = a[64:SC_ROWS] + pos_ref[SPLICE:] + typ
  o_ref[0, SPLICE:, :] = _ln(seg3, gam, bet)


def _sc_call(ids_q, word_emb):
  mesh = plsc.VectorSubcoreMesh(core_axis_name="c", subcore_axis_name="s")
  return pl.kernel(
      _sc_gather,
      out_type=jax.ShapeDtypeStruct((QB * SC_ROWS, HID), jnp.float32),
      mesh=mesh,
      scratch_types=[
          pltpu.VMEM((GCHUNK,), jnp.int32),
          pltpu.VMEM((GCHUNK,), jnp.int32),
          pltpu.VMEM((GCHUNK, HID), jnp.float32),
          pltpu.VMEM((GCHUNK, HID), jnp.float32),
          pltpu.SemaphoreType.DMA,
          pltpu.SemaphoreType.DMA,
          pltpu.SemaphoreType.DMA,
          pltpu.SemaphoreType.DMA,
      ],
  )(ids_q, word_emb)


def _tc_call(dummy, a_q, qi, vt_f, vt_p, pos_emb, typ, gam, bet, alias):
  return pl.pallas_call(
      _tc_body,
      grid=(QB,),
      in_specs=[
          pl.BlockSpec(memory_space=pl.ANY),
          pl.BlockSpec((1, SC_ROWS, HID), lambda b: (b, 0, 0)),
          pl.BlockSpec((LV, QB, HID), lambda b, q=qi: (0, q, 0)),
          pl.BlockSpec((LV, QB, HID), lambda b, q=qi: (0, q, 0)),
          pl.BlockSpec((S, HID), lambda b: (0, 0)),
          pl.BlockSpec((1, HID), lambda b: (0, 0)),
          pl.BlockSpec((1, HID), lambda b: (0, 0)),
          pl.BlockSpec((1, HID), lambda b: (0, 0)),
      ],
      out_specs=pl.BlockSpec((1, S, HID), lambda b, q=qi: (b + q * QB, 0, 0)),
      out_shape=jax.ShapeDtypeStruct((B, S, HID), jnp.float32),
      input_output_aliases={0: 0} if alias else {},
  )(dummy, a_q, vt_f, vt_p, pos_emb, typ, gam, bet)


@jax.jit
def kernel(vis_feats, vis_pe, input_ids, word_emb, pos_emb, type_emb,
           ln_gamma, ln_beta):
  ids = input_ids.reshape(-1).astype(jnp.int32)

  gs = [_sc_call(ids[qi * QB * S:(qi + 1) * QB * S], word_emb)
        for qi in range(NSPLIT)]

  typ = type_emb[0:1]
  gam = ln_gamma.reshape(1, HID)
  bet = ln_beta.reshape(1, HID)
  vt_f = vis_feats.transpose(1, 0, 2)
  vt_p = vis_pe.transpose(1, 0, 2)

  out = None
  for qi in range(NSPLIT):
    a_q = gs[qi].reshape(QB, SC_ROWS, HID)
    dummy = a_q if out is None else out
    out = _tc_call(dummy, a_q, qi, vt_f, vt_p, pos_emb, typ, gam, bet,
                   alias=out is not None)
  return out


# speedup vs baseline: 3.8767x; 1.0215x over previous
import jax
import jax.numpy as jnp
from jax import lax
from jax.experimental import pallas as pl
from jax.experimental.pallas import tpu as pltpu
import jax.experimental.pallas.tpu_sc as plsc

VOCAB = 30522
HID = 768
B = 32
S = 512
LV = 100
EPS = 1e-5

NSPLIT = 4
QB = B // NSPLIT
GCHUNK = 32
SPLICE = 128
TCG = 2
NCH = 14
SC_ROWS = NCH * GCHUNK


def _sc_gather(ids_hbm, word_hbm, out_hbm, idx_a, idx_b, rows_a, rows_b,
               gsem_a, gsem_b, ssem_a, ssem_b):
  wid = lax.axis_index("s") * 2 + lax.axis_index("c")
  b_loc = wid // NSPLIT
  q = wid % NSPLIT
  start = q * 4 - jnp.maximum(q - 2, 0)
  last = q * 4 + 3 - jnp.maximum(q - 1, 0)

  idx_v = (idx_a, idx_b)
  rows_v = (rows_a, rows_b)
  gsem = (gsem_a, gsem_b)
  ssem = (ssem_a, ssem_b)
  gd = [None, None]
  st = [None, None]
  for i in range(4):
    p = i % 2
    if st[p] is not None:
      st[p].wait()
    ci = jnp.minimum(start + i, last)
    c = ci + 2 * (ci >= 1).astype(jnp.int32)
    pltpu.sync_copy(ids_hbm.at[pl.ds(b_loc * S + c * GCHUNK, GCHUNK)],
                    idx_v[p])
    gd[p] = pltpu.async_copy(word_hbm.at[idx_v[p]], rows_v[p], gsem[p])
    pp = (i - 1) % 2
    if gd[pp] is not None and i >= 1:
      gd[pp].wait()
      st[pp] = pltpu.async_copy(
          rows_v[pp],
          out_hbm.at[pl.ds(b_loc * SC_ROWS + _ci_prev(start, last, i - 1)
                           * GCHUNK, GCHUNK)],
          ssem[pp])
  gd[3 % 2].wait()
  st[3 % 2] = pltpu.async_copy(
      rows_v[3 % 2],
      out_hbm.at[pl.ds(b_loc * SC_ROWS + _ci_prev(start, last, 3)
                       * GCHUNK, GCHUNK)],
      ssem[3 % 2])
  st[0].wait()
  st[1].wait()


def _ci_prev(start, last, i):
  return jnp.minimum(start + i, last)


def _ln(emb, gam, bet):
  u = jnp.mean(emb, axis=1, keepdims=True)
  d = emb - u
  var = jnp.mean(d * d, axis=1, keepdims=True)
  return gam * (d * lax.rsqrt(var + EPS)) + bet


def _tc_body(dummy_ref, a_ref, visf_ref, vispe_ref, pos_ref, typ_ref,
             gam_ref, bet_ref, o_ref):
  g = pl.program_id(0)
  typ = typ_ref[...]
  gam = gam_ref[...]
  bet = bet_ref[...]
  r = lax.broadcasted_iota(jnp.int32, (SPLICE, 1), 0)
  c = lax.broadcasted_iota(jnp.int32, (1, LV), 1)
  perm = (r == c + 1).astype(jnp.float32)
  m0 = jnp.logical_and(r[:32] >= 1, r[:32] <= LV)
  m2 = r[96:SPLICE] <= LV
  for j in range(TCG):
    a = a_ref[j]
    b = g * TCG + j
    vs = visf_ref[:, b, :] + vispe_ref[:, b, :]
    vss = jax.lax.dot(perm, vs,
                      preferred_element_type=jnp.float32)
    seg0 = jnp.where(m0, vss[:32], a[0:32] + pos_ref[0:32]) + typ
    o_ref[j, 0:32, :] = _ln(seg0, gam, bet)
    seg1 = vss[32:96] + typ
    o_ref[j, 32:96, :] = _ln(seg1, gam, bet)
    seg2 = jnp.where(m2, vss[96:SPLICE], a[32:64] + pos_ref[96:SPLICE]) + typ
    o_ref[j, 96:SPLICE, :] = _ln(seg2, gam, bet)
    seg3 = a[64:SC_ROWS] + pos_ref[SPLICE:] + typ
    o_ref[j, SPLICE:, :] = _ln(seg3, gam, bet)


def _sc_call(ids_q, word_emb):
  mesh = plsc.VectorSubcoreMesh(core_axis_name="c", subcore_axis_name="s")
  return pl.kernel(
      _sc_gather,
      out_type=jax.ShapeDtypeStruct((QB * SC_ROWS, HID), jnp.float32),
      mesh=mesh,
      scratch_types=[
          pltpu.VMEM((GCHUNK,), jnp.int32),
          pltpu.VMEM((GCHUNK,), jnp.int32),
          pltpu.VMEM((GCHUNK, HID), jnp.float32),
          pltpu.VMEM((GCHUNK, HID), jnp.float32),
          pltpu.SemaphoreType.DMA,
          pltpu.SemaphoreType.DMA,
          pltpu.SemaphoreType.DMA,
          pltpu.SemaphoreType.DMA,
      ],
  )(ids_q, word_emb)


def _tc_call(dummy, a_q, qi, vt_f, vt_p, pos_emb, typ, gam, bet, alias):
  return pl.pallas_call(
      _tc_body,
      grid=(QB // TCG,),
      in_specs=[
          pl.BlockSpec(memory_space=pl.ANY),
          pl.BlockSpec((TCG, SC_ROWS, HID), lambda b: (b, 0, 0)),
          pl.BlockSpec((LV, QB, HID), lambda b, q=qi: (0, q, 0)),
          pl.BlockSpec((LV, QB, HID), lambda b, q=qi: (0, q, 0)),
          pl.BlockSpec((S, HID), lambda b: (0, 0)),
          pl.BlockSpec((1, HID), lambda b: (0, 0)),
          pl.BlockSpec((1, HID), lambda b: (0, 0)),
          pl.BlockSpec((1, HID), lambda b: (0, 0)),
      ],
      out_specs=pl.BlockSpec((TCG, S, HID),
                             lambda b, q=qi: (b + q * (QB // TCG), 0, 0)),
      out_shape=jax.ShapeDtypeStruct((B, S, HID), jnp.float32),
      input_output_aliases={0: 0} if alias else {},
  )(dummy, a_q, vt_f, vt_p, pos_emb, typ, gam, bet)


@jax.jit
def kernel(vis_feats, vis_pe, input_ids, word_emb, pos_emb, type_emb,
           ln_gamma, ln_beta):
  ids = input_ids.reshape(-1).astype(jnp.int32)

  gs = [_sc_call(ids[qi * QB * S:(qi + 1) * QB * S], word_emb)
        for qi in range(NSPLIT)]

  typ = type_emb[0:1]
  gam = ln_gamma.reshape(1, HID)
  bet = ln_beta.reshape(1, HID)
  vt_f = vis_feats.transpose(1, 0, 2)
  vt_p = vis_pe.transpose(1, 0, 2)

  out = None
  for qi in range(NSPLIT):
    a_q = gs[qi].reshape(QB, SC_ROWS, HID)
    dummy = a_q if out is None else out
    out = _tc_call(dummy, a_q, qi, vt_f, vt_p, pos_emb, typ, gam, bet,
                   alias=out is not None)
  return out


# speedup vs baseline: 4.0257x; 1.0384x over previous
import jax
import jax.numpy as jnp
from jax import lax
from jax.experimental import pallas as pl
from jax.experimental.pallas import tpu as pltpu
import jax.experimental.pallas.tpu_sc as plsc

VOCAB = 30522
HID = 768
B = 32
S = 512
LV = 100
EPS = 1e-5

NSPLIT = 4
QB = B // NSPLIT
GCHUNK = 32
SPLICE = 128
TCG = 4
NCH = 14
SC_ROWS = NCH * GCHUNK


def _sc_gather(ids_hbm, word_hbm, out_hbm, idx_a, idx_b, rows_a, rows_b,
               gsem_a, gsem_b, ssem_a, ssem_b):
  wid = lax.axis_index("s") * 2 + lax.axis_index("c")
  b_loc = wid // NSPLIT
  q = wid % NSPLIT
  start = q * 4 - jnp.maximum(q - 2, 0)
  last = q * 4 + 3 - jnp.maximum(q - 1, 0)

  idx_v = (idx_a, idx_b)
  rows_v = (rows_a, rows_b)
  gsem = (gsem_a, gsem_b)
  ssem = (ssem_a, ssem_b)
  gd = [None, None]
  st = [None, None]
  for i in range(4):
    p = i % 2
    if st[p] is not None:
      st[p].wait()
    ci = jnp.minimum(start + i, last)
    c = ci + 2 * (ci >= 1).astype(jnp.int32)
    pltpu.sync_copy(ids_hbm.at[pl.ds(b_loc * S + c * GCHUNK, GCHUNK)],
                    idx_v[p])
    gd[p] = pltpu.async_copy(word_hbm.at[idx_v[p]], rows_v[p], gsem[p])
    pp = (i - 1) % 2
    if gd[pp] is not None and i >= 1:
      gd[pp].wait()
      st[pp] = pltpu.async_copy(
          rows_v[pp],
          out_hbm.at[pl.ds(b_loc * SC_ROWS + _ci_prev(start, last, i - 1)
                           * GCHUNK, GCHUNK)],
          ssem[pp])
  gd[3 % 2].wait()
  st[3 % 2] = pltpu.async_copy(
      rows_v[3 % 2],
      out_hbm.at[pl.ds(b_loc * SC_ROWS + _ci_prev(start, last, 3)
                       * GCHUNK, GCHUNK)],
      ssem[3 % 2])
  st[0].wait()
  st[1].wait()


def _ci_prev(start, last, i):
  return jnp.minimum(start + i, last)


def _ln(emb, gam, bet):
  u = jnp.mean(emb, axis=1, keepdims=True)
  d = emb - u
  var = jnp.mean(d * d, axis=1, keepdims=True)
  return gam * (d * lax.rsqrt(var + EPS)) + bet


def _tc_body(dummy_ref, a_ref, visf_ref, vispe_ref, pos_ref, typ_ref,
             gam_ref, bet_ref, o_ref):
  g = pl.program_id(0)
  typ = typ_ref[...]
  gam = gam_ref[...]
  bet = bet_ref[...]
  r = lax.broadcasted_iota(jnp.int32, (SPLICE, 1), 0)
  c = lax.broadcasted_iota(jnp.int32, (1, LV), 1)
  perm = (r == c + 1).astype(jnp.float32)
  m0 = jnp.logical_and(r[:32] >= 1, r[:32] <= LV)
  m2 = r[96:SPLICE] <= LV
  for j in range(TCG):
    a = a_ref[j]
    b = g * TCG + j
    vs = visf_ref[:, b, :] + vispe_ref[:, b, :]
    vss = jax.lax.dot(perm, vs,
                      preferred_element_type=jnp.float32)
    seg0 = jnp.where(m0, vss[:32], a[0:32] + pos_ref[0:32]) + typ
    o_ref[j, 0:32, :] = _ln(seg0, gam, bet)
    seg1 = vss[32:96] + typ
    o_ref[j, 32:96, :] = _ln(seg1, gam, bet)
    seg2 = jnp.where(m2, vss[96:SPLICE], a[32:64] + pos_ref[96:SPLICE]) + typ
    o_ref[j, 96:SPLICE, :] = _ln(seg2, gam, bet)
    seg3 = a[64:SC_ROWS] + pos_ref[SPLICE:] + typ
    o_ref[j, SPLICE:, :] = _ln(seg3, gam, bet)


def _sc_call(ids_q, word_emb):
  mesh = plsc.VectorSubcoreMesh(core_axis_name="c", subcore_axis_name="s")
  return pl.kernel(
      _sc_gather,
      out_type=jax.ShapeDtypeStruct((QB * SC_ROWS, HID), jnp.float32),
      mesh=mesh,
      scratch_types=[
          pltpu.VMEM((GCHUNK,), jnp.int32),
          pltpu.VMEM((GCHUNK,), jnp.int32),
          pltpu.VMEM((GCHUNK, HID), jnp.float32),
          pltpu.VMEM((GCHUNK, HID), jnp.float32),
          pltpu.SemaphoreType.DMA,
          pltpu.SemaphoreType.DMA,
          pltpu.SemaphoreType.DMA,
          pltpu.SemaphoreType.DMA,
      ],
  )(ids_q, word_emb)


def _tc_call(dummy, a_q, qi, vt_f, vt_p, pos_emb, typ, gam, bet, alias):
  return pl.pallas_call(
      _tc_body,
      grid=(QB // TCG,),
      in_specs=[
          pl.BlockSpec(memory_space=pl.ANY),
          pl.BlockSpec((TCG, SC_ROWS, HID), lambda b: (b, 0, 0)),
          pl.BlockSpec((LV, QB, HID), lambda b, q=qi: (0, q, 0)),
          pl.BlockSpec((LV, QB, HID), lambda b, q=qi: (0, q, 0)),
          pl.BlockSpec((S, HID), lambda b: (0, 0)),
          pl.BlockSpec((1, HID), lambda b: (0, 0)),
          pl.BlockSpec((1, HID), lambda b: (0, 0)),
          pl.BlockSpec((1, HID), lambda b: (0, 0)),
      ],
      out_specs=pl.BlockSpec((TCG, S, HID),
                             lambda b, q=qi: (b + q * (QB // TCG), 0, 0)),
      out_shape=jax.ShapeDtypeStruct((B, S, HID), jnp.float32),
      input_output_aliases={0: 0} if alias else {},
  )(dummy, a_q, vt_f, vt_p, pos_emb, typ, gam, bet)


@jax.jit
def kernel(vis_feats, vis_pe, input_ids, word_emb, pos_emb, type_emb,
           ln_gamma, ln_beta):
  ids = input_ids.reshape(-1).astype(jnp.int32)

  gs = [_sc_call(ids[qi * QB * S:(qi + 1) * QB * S], word_emb)
        for qi in range(NSPLIT)]

  typ = type_emb[0:1]
  gam = ln_gamma.reshape(1, HID)
  bet = ln_beta.reshape(1, HID)
  vt_f = vis_feats.transpose(1, 0, 2)
  vt_p = vis_pe.transpose(1, 0, 2)

  out = None
  for qi in range(NSPLIT):
    a_q = gs[qi].reshape(QB, SC_ROWS, HID)
    dummy = a_q if out is None else out
    out = _tc_call(dummy, a_q, qi, vt_f, vt_p, pos_emb, typ, gam, bet,
                   alias=out is not None)
  return out
